# baseline (speedup 1.0000x reference)
import functools

import jax
import jax.numpy as jnp
from jax.experimental import pallas as pl
from jax.experimental.pallas import tpu as pltpu


def _rmsnorm_body(x_ref, w_ref, o_ref, *, eps, inv_n):
    x = x_ref[...].astype(jnp.float32)
    ss = jnp.sum(x * x, axis=1, keepdims=True)
    ss = jnp.sum(ss, axis=2, keepdims=True)
    inv = jax.lax.rsqrt(ss * inv_n + eps)
    o_ref[...] = (x * (inv * w_ref[...])).astype(o_ref.dtype)


def _pick_rows(batch, row_bytes):
    budget = 2 << 20
    best = 1
    for t in range(1, batch + 1):
        if batch % t == 0 and t * row_bytes <= budget and batch // t >= 8:
            best = t
    return best


def kernel(x, weight, eps=1e-5):
    orig_shape = x.shape
    B, C = orig_shape[0], orig_shape[1]
    spatial = 1
    for s in orig_shape[2:]:
        spatial *= s
    N = C * spatial
    itemsize = jnp.dtype(x.dtype).itemsize

    xt = jnp.transpose(x, (0, 2, 3, 1)).reshape(B, spatial, C)
    w3 = weight.astype(x.dtype).reshape(1, 1, C)

    tb = _pick_rows(B, N * itemsize)
    grid = (B // tb,)
    block_bytes = tb * N * itemsize
    vmem_limit = int(max(4 * block_bytes + (4 << 20), 56 << 20))

    out = pl.pallas_call(
        functools.partial(_rmsnorm_body, eps=float(eps), inv_n=1.0 / N),
        out_shape=jax.ShapeDtypeStruct((B, spatial, C), x.dtype),
        grid=grid,
        in_specs=[
            pl.BlockSpec((tb, spatial, C), lambda b: (b, 0, 0)),
            pl.BlockSpec((1, 1, C), lambda b: (0, 0, 0)),
        ],
        out_specs=pl.BlockSpec((tb, spatial, C), lambda b: (b, 0, 0)),
        compiler_params=pltpu.CompilerParams(
            dimension_semantics=("parallel",),
            vmem_limit_bytes=vmem_limit,
        ),
        cost_estimate=pl.CostEstimate(
            flops=4 * B * N,
            transcendentals=B,
            bytes_accessed=2 * B * N * itemsize + C * itemsize,
        ),
    )(xt, w3)
    return jnp.transpose(out.reshape(B, *orig_shape[2:], C),
                         (0, 3, 1, 2))


# speedup vs baseline: 1.1113x; 1.1113x over previous
import functools

import jax
import jax.numpy as jnp
from jax.experimental import pallas as pl
from jax.experimental.pallas import tpu as pltpu


def _rmsnorm_body(x_ref, w_ref, o_ref, *, eps, inv_n):
    x = x_ref[...].astype(jnp.float32)
    ss = jnp.sum(x * x, axis=1, keepdims=True)
    ss = jnp.sum(ss, axis=2, keepdims=True)
    inv = jax.lax.rsqrt(ss * inv_n + eps)
    o_ref[...] = (x * (inv * w_ref[...])).astype(o_ref.dtype)


def _pick_rows(batch, row_bytes):
    budget = 8 << 20
    best = 1
    for t in range(1, batch + 1):
        if batch % t == 0 and t * row_bytes <= budget and batch // t >= 8:
            best = t
    return best


def kernel(x, weight, eps=1e-5):
    orig_shape = x.shape
    B, C = orig_shape[0], orig_shape[1]
    spatial = 1
    for s in orig_shape[2:]:
        spatial *= s
    N = C * spatial
    itemsize = jnp.dtype(x.dtype).itemsize

    xt = jnp.transpose(x, (0, 2, 3, 1)).reshape(B, spatial, C)
    w3 = weight.astype(x.dtype).reshape(1, 1, C)

    tb = _pick_rows(B, N * itemsize)
    grid = (B // tb,)
    block_bytes = tb * N * itemsize
    vmem_limit = int(min(4 * block_bytes + (4 << 20), 24 << 20))

    out = pl.pallas_call(
        functools.partial(_rmsnorm_body, eps=float(eps), inv_n=1.0 / N),
        out_shape=jax.ShapeDtypeStruct((B, spatial, C), x.dtype),
        grid=grid,
        in_specs=[
            pl.BlockSpec((tb, spatial, C), lambda b: (b, 0, 0)),
            pl.BlockSpec((1, 1, C), lambda b: (0, 0, 0),
                         pipeline_mode=pl.Buffered(1)),
        ],
        out_specs=pl.BlockSpec((tb, spatial, C), lambda b: (b, 0, 0)),
        compiler_params=pltpu.CompilerParams(
            dimension_semantics=("parallel",),
            vmem_limit_bytes=vmem_limit,
        ),
        cost_estimate=pl.CostEstimate(
            flops=4 * B * N,
            transcendentals=B,
            bytes_accessed=2 * B * N * itemsize + C * itemsize,
        ),
    )(xt, w3)
    return jnp.transpose(out.reshape(B, *orig_shape[2:], C),
                         (0, 3, 1, 2))
